# Initial kernel scaffold; baseline (speedup 1.0000x reference)
import jax
import jax.numpy as jnp
from jax.experimental import pallas as pl


def kernel(x, edge_index, batch, W1, b1, W2, b2, W3, b3, Wl, bl):
    raise NotImplementedError("write your pallas kernel here")


# speedup vs baseline: 16.4951x; 16.4951x over previous
import functools

import jax
import jax.numpy as jnp
from jax import lax
from jax.experimental import pallas as pl
from jax.experimental.pallas import tpu as pltpu
from jax.experimental.pallas import tpu_sc as plsc

N_NODES = 50000
N_EDGES = 800000
N_GRAPHS = 512

N_PAD = 51200
E_PAD = 802816
G_PAD = 520
NTILE = 16
ROWS_PT = N_PAD // NTILE
ECH = E_PAD // 128
CH_PT = ECH // NTILE
CH_PT_HALF = ECH // (2 * NTILE)
BLK = 4
FH = 32

_mesh = plsc.VectorSubcoreMesh(core_axis_name="c", subcore_axis_name="s")
_sc_params = pltpu.CompilerParams(use_tc_tiling_on_sc=False)
f32 = jnp.float32
i32 = jnp.int32


def _fill(ref, rows, value):
    width = ref.shape[1]
    v = jnp.full((16,), value, f32)

    def body(j, _):
        for w in range(width // 16):
            ref[j, pl.ds(w * 16, 16)] = v
        return 0

    lax.fori_loop(0, rows, body, 0)


def _deg_body(col2, d0, d1, acc, cv, ob):
    c = lax.axis_index("c")
    sid = lax.axis_index("s")
    _fill(ob, 128, 0.0)
    for q in range(25):
        pltpu.sync_copy(ob, acc.at[pl.ds(sid * ROWS_PT + q * 128, 128)])
    _fill(ob, 128, 1.0)
    plsc.subcore_barrier()
    base = c * (ECH // 2) + sid * CH_PT_HALF

    def body(i, _):
        pltpu.sync_copy(col2.at[pl.ds(base + i * 4, 4)], cv)
        for k in range(4):
            pltpu.sync_copy(ob, acc.at[cv.at[k]], add=True)
        return 0

    lax.fori_loop(0, CH_PT_HALF // 4, body, 0)
    plsc.subcore_barrier()
    sl = pl.ds(sid * ROWS_PT, ROWS_PT)

    @pl.when(c == 0)
    def _():
        pltpu.sync_copy(acc.at[sl], d0.at[sl])

    @pl.when(c == 1)
    def _():
        pltpu.sync_copy(acc.at[sl], d1.at[sl])


_deg = pl.kernel(
    _deg_body,
    out_type=[jax.ShapeDtypeStruct((N_PAD, 16), f32),
              jax.ShapeDtypeStruct((N_PAD, 16), f32)],
    mesh=_mesh,
    compiler_params=_sc_params,
    scratch_types=[
        pltpu.VMEM_SHARED((N_PAD, 16), f32),
        pltpu.VMEM((4, 128), i32),
        pltpu.VMEM((128, 16), f32),
    ],
)


def _spmm_body(h0, h1, row2, col2, a0, a1, acc, rv, cv, gb, sem_g, sem_s):
    c = lax.axis_index("c")
    sid = lax.axis_index("s")

    def run(h_hbm, out_hbm):
        _fill(gb[0], 128, 0.0)
        for q in range(25):
            pltpu.sync_copy(gb[0], acc.at[pl.ds(sid * ROWS_PT + q * 128, 128)])
        plsc.subcore_barrier()
        base = sid * CH_PT

        def body(b, _):
            pltpu.sync_copy(row2.at[pl.ds(base + b * BLK, BLK)], rv)
            pltpu.sync_copy(col2.at[pl.ds(base + b * BLK, BLK)], cv)
            gets = [pltpu.async_copy(h_hbm.at[rv.at[k]], gb[k], sem_g)
                    for k in range(BLK)]
            for d in gets:
                d.wait()
            puts = [pltpu.async_copy(gb[k], acc.at[cv.at[k]], sem_s, add=True)
                    for k in range(BLK)]
            for d in puts:
                d.wait()
            return 0

        lax.fori_loop(0, CH_PT // BLK, body, 0)
        plsc.subcore_barrier()
        sl = pl.ds(sid * ROWS_PT, ROWS_PT)
        pltpu.sync_copy(acc.at[sl], out_hbm.at[sl])

    @pl.when(c == 0)
    def _():
        run(h0, a0)

    @pl.when(c == 1)
    def _():
        run(h1, a1)


_spmm = pl.kernel(
    _spmm_body,
    out_type=[jax.ShapeDtypeStruct((N_PAD, FH), f32),
              jax.ShapeDtypeStruct((N_PAD, FH), f32)],
    mesh=_mesh,
    compiler_params=_sc_params,
    scratch_types=[
        pltpu.VMEM_SHARED((N_PAD, FH), f32),
        pltpu.VMEM((BLK, 128), i32),
        pltpu.VMEM((BLK, 128), i32),
        [pltpu.VMEM((128, FH), f32) for _ in range(BLK)],
        pltpu.SemaphoreType.DMA,
        pltpu.SemaphoreType.DMA,
    ],
)


def _pool_body(h30, h31, b2, p0, p1, cnt, accp, accc, bv, hb, ob, zb32, zb16):
    c = lax.axis_index("c")
    sid = lax.axis_index("s")
    _fill(ob, 128, 1.0)

    @pl.when(sid < 13)
    def _():
        _fill(zb32, 40, 0.0)
        pltpu.sync_copy(zb32, accp.at[pl.ds(sid * 40, 40)])

    @pl.when((c == 0) & (sid < 13))
    def _():
        _fill(zb16, 40, 0.0)
        pltpu.sync_copy(zb16, accc.at[pl.ds(sid * 40, 40)])

    plsc.subcore_barrier()

    @pl.when(c == 0)
    def _():
        def body(i, _):
            pltpu.sync_copy(b2.at[pl.ds(sid * 25 + i, 1)], bv)
            pltpu.sync_copy(h30.at[pl.ds(sid * ROWS_PT + i * 128, 128)], hb)
            pltpu.sync_copy(hb, accp.at[bv.at[0]], add=True)
            pltpu.sync_copy(ob, accc.at[bv.at[0]], add=True)
            return 0

        lax.fori_loop(0, 25, body, 0)

    @pl.when(c == 1)
    def _():
        def body(i, _):
            pltpu.sync_copy(b2.at[pl.ds(sid * 25 + i, 1)], bv)
            pltpu.sync_copy(h31.at[pl.ds(sid * ROWS_PT + i * 128, 128)], hb)
            pltpu.sync_copy(hb, accp.at[bv.at[0]], add=True)
            return 0

        lax.fori_loop(0, 25, body, 0)

    plsc.subcore_barrier()
    sl = pl.ds(sid * 40, 40)

    @pl.when((c == 0) & (sid < 13))
    def _():
        pltpu.sync_copy(accp.at[sl], p0.at[sl])
        pltpu.sync_copy(accc.at[sl], cnt.at[sl])

    @pl.when((c == 1) & (sid < 13))
    def _():
        pltpu.sync_copy(accp.at[sl], p1.at[sl])


_pool = pl.kernel(
    _pool_body,
    out_type=[jax.ShapeDtypeStruct((G_PAD, FH), f32),
              jax.ShapeDtypeStruct((G_PAD, FH), f32),
              jax.ShapeDtypeStruct((G_PAD, 16), f32)],
    mesh=_mesh,
    compiler_params=_sc_params,
    scratch_types=[
        pltpu.VMEM_SHARED((G_PAD, FH), f32),
        pltpu.VMEM_SHARED((G_PAD, 16), f32),
        pltpu.VMEM((1, 128), i32),
        pltpu.VMEM((128, FH), f32),
        pltpu.VMEM((128, 16), f32),
        pltpu.VMEM((40, FH), f32),
        pltpu.VMEM((40, 16), f32),
    ],
)


def _prep_tc(x_ref, d0_ref, d1_ref, w_ref, s_ref, hs0_ref, hs1_ref):
    deg = d0_ref[:, 0] + d1_ref[:, 0] + 1.0
    s = lax.rsqrt(deg)[:, None]
    s_ref[...] = s
    hp = jnp.dot(x_ref[...], w_ref[...], preferred_element_type=f32)
    hs = hp * s
    hs0_ref[...] = hs[:, :FH]
    hs1_ref[...] = hs[:, FH:]


def _mid_tc(a0, a1, hs0, hs1, s_ref, b_ref, w_ref, o0, o1):
    s = s_ref[...]
    t = jnp.concatenate([a0[...] + hs0[...], a1[...] + hs1[...]], axis=1)
    h = jnp.maximum(t * s + b_ref[...], 0.0)
    hs = jnp.dot(h, w_ref[...], preferred_element_type=f32) * s
    o0[...] = hs[:, :FH]
    o1[...] = hs[:, FH:]


def _last_tc(a0, a1, hs0, hs1, s_ref, b_ref, o0, o1):
    s = s_ref[...]
    t = jnp.concatenate([a0[...] + hs0[...], a1[...] + hs1[...]], axis=1)
    h = t * s + b_ref[...]
    o0[...] = h[:, :FH]
    o1[...] = h[:, FH:]


def _final_tc(p0, p1, cnt_ref, wl_ref, bl_ref, out_ref):
    sums = jnp.concatenate([p0[...], p1[...]], axis=1)[:N_GRAPHS]
    c = jnp.maximum(cnt_ref[:N_GRAPHS, 0:1], 1.0)
    z = jnp.dot(sums / c, wl_ref[...], preferred_element_type=f32) + bl_ref[...]
    out_ref[...] = 1.0 / (1.0 + jnp.exp(-z))


_RB = 512
_GRID = N_PAD // _RB


def _rows_spec(w):
    return pl.BlockSpec((_RB, w), lambda i: (i, 0))


def _full_spec(shape):
    return pl.BlockSpec(shape, lambda i: tuple(0 for _ in shape))


_prep = pl.pallas_call(
    _prep_tc,
    grid=(_GRID,),
    in_specs=[_rows_spec(32), _rows_spec(16), _rows_spec(16), _full_spec((32, 64))],
    out_specs=[_rows_spec(1), _rows_spec(FH), _rows_spec(FH)],
    out_shape=[jax.ShapeDtypeStruct((N_PAD, 1), f32),
               jax.ShapeDtypeStruct((N_PAD, FH), f32),
               jax.ShapeDtypeStruct((N_PAD, FH), f32)],
)

_mid = pl.pallas_call(
    _mid_tc,
    grid=(_GRID,),
    in_specs=[_rows_spec(FH), _rows_spec(FH), _rows_spec(FH), _rows_spec(FH),
              _rows_spec(1), _full_spec((1, 64)), _full_spec((64, 64))],
    out_specs=[_rows_spec(FH), _rows_spec(FH)],
    out_shape=[jax.ShapeDtypeStruct((N_PAD, FH), f32),
               jax.ShapeDtypeStruct((N_PAD, FH), f32)],
)

_last = pl.pallas_call(
    _last_tc,
    grid=(_GRID,),
    in_specs=[_rows_spec(FH), _rows_spec(FH), _rows_spec(FH), _rows_spec(FH),
              _rows_spec(1), _full_spec((1, 64))],
    out_specs=[_rows_spec(FH), _rows_spec(FH)],
    out_shape=[jax.ShapeDtypeStruct((N_PAD, FH), f32),
               jax.ShapeDtypeStruct((N_PAD, FH), f32)],
)

_final = pl.pallas_call(
    _final_tc,
    out_shape=jax.ShapeDtypeStruct((N_GRAPHS, 1), f32),
)


def kernel(x, edge_index, batch, W1, b1, W2, b2, W3, b3, Wl, bl):
    x_p = jnp.zeros((N_PAD, 32), f32).at[:N_NODES, :27].set(x)
    w1_p = jnp.zeros((32, 64), f32).at[:27].set(W1)
    row_p = jnp.concatenate(
        [edge_index[0], jnp.zeros((E_PAD - N_EDGES,), i32)]).reshape(ECH, 128)
    col_p = jnp.concatenate(
        [edge_index[1],
         jnp.full((E_PAD - N_EDGES,), N_PAD - 1, i32)]).reshape(ECH, 128)
    batch_p = jnp.concatenate(
        [batch, jnp.full((N_PAD - N_NODES,), N_GRAPHS, i32)]).reshape(400, 128)

    d0, d1 = _deg(col_p)
    s, hs0, hs1 = _prep(x_p, d0, d1, w1_p)
    a0, a1 = _spmm(hs0, hs1, row_p, col_p)
    hs0, hs1 = _mid(a0, a1, hs0, hs1, s, b1.reshape(1, 64), W2)
    a0, a1 = _spmm(hs0, hs1, row_p, col_p)
    hs0, hs1 = _mid(a0, a1, hs0, hs1, s, b2.reshape(1, 64), W3)
    a0, a1 = _spmm(hs0, hs1, row_p, col_p)
    h30, h31 = _last(a0, a1, hs0, hs1, s, b3.reshape(1, 64))
    p0, p1, cnt = _pool(h30, h31, batch_p)
    return _final(p0, p1, cnt, Wl, bl.reshape(1, 1))
